# Initial kernel scaffold; baseline (speedup 1.0000x reference)
import jax
import jax.numpy as jnp
from jax.experimental import pallas as pl


def kernel(x, w1, w2, in_ids, out_ids):
    raise NotImplementedError("write your pallas kernel here")


# speedup vs baseline: 2.7100x; 2.7100x over previous
import jax
import jax.numpy as jnp
from jax import lax
from jax.experimental import pallas as pl
from jax.experimental.pallas import tpu as pltpu
from jax.experimental.pallas import tpu_sc as plsc

B = 128
I = 10000
O = 10000
H = 4
E = 160000

EB = 128
NBLK = E // EB
NSUB = 16
NCORE = 2
ROWS_PER_SUB = 624
TAIL_ROWS = O - NSUB * ROWS_PER_SUB


def _splat(vec, k):
    idx = jnp.full((16, 1), k, jnp.int32)
    dn = lax.GatherDimensionNumbers(
        offset_dims=(), collapsed_slice_dims=(0,), start_index_map=(0,))
    return lax.gather(vec, idx, dn, slice_sizes=(1,),
                      mode=lax.GatherScatterMode.PROMISE_IN_BOUNDS)


def _sc_body(xt, inids, outids, w1f, zeros_hbm, out_hbm,
             inbuf, outbuf, wbuf, gbuf, sbuf, acc, sem):
    cid = lax.axis_index("c")
    sid = lax.axis_index("s")
    r0 = sid * ROWS_PER_SUB

    for jpass in range(H):
        @pl.when(cid == jpass // 2)
        def _(jpass=jpass):
            pltpu.sync_copy(zeros_hbm.at[pl.ds(r0, ROWS_PER_SUB)],
                            acc.at[pl.ds(r0, ROWS_PER_SUB)])

            @pl.when(sid == NSUB - 1)
            def _():
                pltpu.sync_copy(
                    zeros_hbm.at[pl.ds(NSUB * ROWS_PER_SUB, TAIL_ROWS)],
                    acc.at[pl.ds(NSUB * ROWS_PER_SUB, TAIL_ROWS)])
            plsc.subcore_barrier()

            @pl.loop(0, (NBLK + NSUB - 1) // NSUB)
            def _(i):
                blk = sid + NSUB * i

                @pl.when(blk < NBLK)
                def _():
                    e0 = blk * EB
                    pltpu.sync_copy(inids.at[pl.ds(e0, EB)], inbuf)
                    pltpu.sync_copy(outids.at[pl.ds(e0, EB)], outbuf)
                    pltpu.sync_copy(w1f.at[pl.ds(e0 * H, EB * H)], wbuf)
                    pltpu.async_copy(xt.at[inbuf], gbuf, sem).wait()

                    @pl.loop(0, EB // 4)
                    def _(g):
                        wv = wbuf[pl.ds(g * 16 * H // 4, 16)]
                        for t in range(4):
                            e = g * 4 + t
                            w = _splat(wv, t * H + jpass)
                            for h in range(8):
                                sbuf[e, pl.ds(h * 16, 16)] = (
                                    gbuf[e, pl.ds(h * 16, 16)] * w)

                    pltpu.sync_copy(sbuf, acc.at[outbuf], add=True)

            plsc.subcore_barrier()
            pltpu.sync_copy(acc.at[pl.ds(r0, ROWS_PER_SUB)],
                            out_hbm.at[jpass, pl.ds(r0, ROWS_PER_SUB)])

            @pl.when(sid == NSUB - 1)
            def _():
                pltpu.sync_copy(
                    acc.at[pl.ds(NSUB * ROWS_PER_SUB, TAIL_ROWS)],
                    out_hbm.at[jpass, pl.ds(NSUB * ROWS_PER_SUB, TAIL_ROWS)])
            plsc.subcore_barrier()


def _sc_accumulate(xt, in_ids, out_ids, w1f, zeros):
    mesh = plsc.VectorSubcoreMesh(core_axis_name="c", subcore_axis_name="s",
                                  num_cores=NCORE, num_subcores=NSUB)
    f = pl.kernel(
        _sc_body,
        out_type=jax.ShapeDtypeStruct((H, O, B), jnp.float32),
        mesh=mesh,
        scratch_types=[
            pltpu.VMEM((EB,), jnp.int32),
            pltpu.VMEM((EB,), jnp.int32),
            pltpu.VMEM((EB * H,), jnp.float32),
            pltpu.VMEM((EB, B), jnp.float32),
            pltpu.VMEM((EB, B), jnp.float32),
            pltpu.VMEM_SHARED((O, B), jnp.float32),
            pltpu.SemaphoreType.DMA,
        ],
    )
    return f(xt, in_ids, out_ids, w1f, zeros)


OB = 1000


def _tc_body(hp_ref, w2_ref, th_ref, yr_ref):
    h = jnp.tanh(hp_ref[...])
    th_ref[...] = h
    w2b = w2_ref[...]
    acc = jnp.zeros((OB, B), jnp.float32)
    for j in range(H):
        acc = acc + h[j] * w2b[:, j:j + 1]
    yr_ref[...] = jnp.tanh(acc)


def _tc_post(hp, w2):
    return pl.pallas_call(
        _tc_body,
        grid=(O // OB,),
        in_specs=[
            pl.BlockSpec((H, OB, B), lambda o: (0, o, 0)),
            pl.BlockSpec((OB, H), lambda o: (o, 0)),
        ],
        out_specs=[
            pl.BlockSpec((H, OB, B), lambda o: (0, o, 0)),
            pl.BlockSpec((OB, B), lambda o: (o, 0)),
        ],
        out_shape=[
            jax.ShapeDtypeStruct((H, O, B), jnp.float32),
            jax.ShapeDtypeStruct((O, B), jnp.float32),
        ],
    )(hp, w2)


@jax.jit
def kernel(x, w1, w2, in_ids, out_ids):
    xt = x.T
    w1f = w1.reshape(-1)
    zeros = jnp.zeros((O, B), jnp.float32)

    hp = _sc_accumulate(xt, in_ids, out_ids, w1f, zeros)
    th, yr = _tc_post(hp, w2)

    hidden = th.transpose(2, 1, 0)
    y = yr.T
    return (y, hidden)
